# baseline (speedup 1.0000x reference)
import jax
import jax.numpy as jnp
from jax.experimental import pallas as pl
from jax.experimental.pallas import tpu as pltpu

_BM = 400
_NBUF = 3


def _outer_kernel(x_ref, adj_ref, w_ref, b_ref, out_ref, support_ref):
    support_ref[...] = jnp.dot(
        x_ref[...], w_ref[...], preferred_element_type=jnp.float32
    )
    n = adj_ref.shape[0]

    def body(adj_blk_ref, out_blk_ref):
        out = jnp.dot(
            adj_blk_ref[...], support_ref[...],
            preferred_element_type=jnp.float32,
        )
        h = jnp.maximum(out + b_ref[...], 0.0)
        m = jnp.max(h, axis=1, keepdims=True)
        lse = m + jnp.log(jnp.sum(jnp.exp(h - m), axis=1, keepdims=True))
        out_blk_ref[...] = h - lse

    pipeline = pltpu.emit_pipeline(
        body,
        grid=(n // _BM,),
        in_specs=[
            pl.BlockSpec((_BM, n), lambda i: (i, 0),
                         pipeline_mode=pl.Buffered(buffer_count=_NBUF)),
        ],
        out_specs=[
            pl.BlockSpec((_BM, out_ref.shape[1]), lambda i: (i, 0)),
        ],
    )
    pipeline(adj_ref, out_ref)


def kernel(x, adj, W, b):
    n, feat = x.shape
    nclass = W.shape[1]
    b2 = b.reshape(1, nclass)
    return pl.pallas_call(
        _outer_kernel,
        in_specs=[
            pl.BlockSpec(memory_space=pltpu.VMEM),
            pl.BlockSpec(memory_space=pl.ANY),
            pl.BlockSpec(memory_space=pltpu.VMEM),
            pl.BlockSpec(memory_space=pltpu.VMEM),
        ],
        out_specs=pl.BlockSpec(memory_space=pl.ANY),
        out_shape=jax.ShapeDtypeStruct((n, nclass), jnp.float32),
        scratch_shapes=[pltpu.VMEM((n, nclass), jnp.float32)],
        compiler_params=pltpu.CompilerParams(
            vmem_limit_bytes=64 * 1024 * 1024,
        ),
    )(x, adj, W, b2)


# speedup vs baseline: 1.0004x; 1.0004x over previous
import jax
import jax.numpy as jnp
from jax.experimental import pallas as pl
from jax.experimental.pallas import tpu as pltpu

_SLOT_ROWS = 256
_NBUF = 4


def _block_schedule(n):
    ramp_up = [64, 128]
    ramp_down = [128, 64]
    mid = n - sum(ramp_up) - sum(ramp_down)
    k, r = divmod(mid, _SLOT_ROWS)
    sizes = ramp_up + [_SLOT_ROWS] * k + ([r] if r else []) + ramp_down
    offs, o = [], 0
    for s in sizes:
        offs.append(o)
        o += s
    return list(zip(offs, sizes))


def _outer_kernel(x_hbm, adj_hbm, w_ref, b_ref, out_ref,
                  x_ref, support_ref, bufs, sems, x_sem):
    n = adj_hbm.shape[0]
    sched = _block_schedule(n)

    def adj_copy(idx):
        off, sz = sched[idx]
        slot = idx % _NBUF
        return pltpu.make_async_copy(
            adj_hbm.at[pl.ds(off, sz), :],
            bufs[slot].at[pl.ds(0, sz), :],
            sems.at[slot],
        )

    for j in range(_NBUF - 1):
        adj_copy(j).start()
    x_copy = pltpu.make_async_copy(x_hbm, x_ref, x_sem)
    x_copy.start()
    x_copy.wait()
    support_ref[...] = jnp.dot(
        x_ref[...], w_ref[...], preferred_element_type=jnp.float32
    )

    b = b_ref[...]
    for idx, (off, sz) in enumerate(sched):
        nxt = idx + _NBUF - 1
        if nxt < len(sched):
            adj_copy(nxt).start()
        adj_copy(idx).wait()
        blk = bufs[idx % _NBUF][pl.ds(0, sz), :]
        out = jnp.dot(
            blk, support_ref[...], preferred_element_type=jnp.float32
        )
        h = jnp.maximum(out + b, 0.0)
        m = jnp.max(h, axis=1, keepdims=True)
        lse = m + jnp.log(jnp.sum(jnp.exp(h - m), axis=1, keepdims=True))
        out_ref[pl.ds(off, sz), :] = h - lse


def kernel(x, adj, W, b):
    n, feat = x.shape
    nclass = W.shape[1]
    b2 = b.reshape(1, nclass)
    return pl.pallas_call(
        _outer_kernel,
        in_specs=[
            pl.BlockSpec(memory_space=pl.ANY),
            pl.BlockSpec(memory_space=pl.ANY),
            pl.BlockSpec(memory_space=pltpu.VMEM),
            pl.BlockSpec(memory_space=pltpu.VMEM),
        ],
        out_specs=pl.BlockSpec(memory_space=pltpu.VMEM),
        out_shape=jax.ShapeDtypeStruct((n, nclass), jnp.float32),
        scratch_shapes=[
            pltpu.VMEM((n, feat), jnp.float32),
            pltpu.VMEM((n, nclass), jnp.float32),
            [pltpu.VMEM((_SLOT_ROWS, n), jnp.float32) for _ in range(_NBUF)],
            pltpu.SemaphoreType.DMA((_NBUF,)),
            pltpu.SemaphoreType.DMA,
        ],
        compiler_params=pltpu.CompilerParams(
            vmem_limit_bytes=64 * 1024 * 1024,
        ),
    )(x, adj, W, b2)


# speedup vs baseline: 1.0166x; 1.0162x over previous
import jax
import jax.numpy as jnp
from jax.experimental import pallas as pl
from jax.experimental.pallas import tpu as pltpu

_SLOT_ROWS = 256
_NBUF = 4


def _block_schedule(n):
    ramp_up = [64, 128]
    ramp_down = [128, 64]
    mid = n - sum(ramp_up) - sum(ramp_down)
    k, r = divmod(mid, _SLOT_ROWS)
    sizes = ramp_up + [_SLOT_ROWS] * k + ([r] if r else []) + ramp_down
    offs, o = [], 0
    for s in sizes:
        offs.append(o)
        o += s
    return list(zip(offs, sizes))


def _outer_kernel(x_hbm, adj_hbm, w_ref, b_ref, out_ref,
                  x_ref, support_ref, bufs, sems, x_sem):
    n = adj_hbm.shape[0]
    sched = _block_schedule(n)

    def adj_copy(idx):
        off, sz = sched[idx]
        slot = idx % _NBUF
        return pltpu.make_async_copy(
            adj_hbm.at[pl.ds(off, sz), :],
            bufs[slot].at[pl.ds(0, sz), :],
            sems.at[slot],
        )

    for j in range(_NBUF - 1):
        adj_copy(j).start()
    x_copy = pltpu.make_async_copy(x_hbm, x_ref, x_sem)
    x_copy.start()
    x_copy.wait()
    support_ref[...] = jnp.dot(
        x_ref[...], w_ref[...], preferred_element_type=jnp.float32
    ).astype(jnp.bfloat16)

    b = b_ref[...]
    for idx, (off, sz) in enumerate(sched):
        nxt = idx + _NBUF - 1
        if nxt < len(sched):
            adj_copy(nxt).start()
        adj_copy(idx).wait()
        blk = bufs[idx % _NBUF][pl.ds(0, sz), :]
        out = jnp.dot(
            blk.astype(jnp.bfloat16), support_ref[...],
            preferred_element_type=jnp.float32,
        )
        h = jnp.maximum(out + b, 0.0)
        m = jnp.max(h, axis=1, keepdims=True)
        lse = m + jnp.log(jnp.sum(jnp.exp(h - m), axis=1, keepdims=True))
        out_ref[pl.ds(off, sz), :] = h - lse


def kernel(x, adj, W, b):
    n, feat = x.shape
    nclass = W.shape[1]
    b2 = b.reshape(1, nclass)
    return pl.pallas_call(
        _outer_kernel,
        in_specs=[
            pl.BlockSpec(memory_space=pl.ANY),
            pl.BlockSpec(memory_space=pl.ANY),
            pl.BlockSpec(memory_space=pltpu.VMEM),
            pl.BlockSpec(memory_space=pltpu.VMEM),
        ],
        out_specs=pl.BlockSpec(memory_space=pltpu.VMEM),
        out_shape=jax.ShapeDtypeStruct((n, nclass), jnp.float32),
        scratch_shapes=[
            pltpu.VMEM((n, feat), jnp.float32),
            pltpu.VMEM((n, nclass), jnp.bfloat16),
            [pltpu.VMEM((_SLOT_ROWS, n), jnp.float32) for _ in range(_NBUF)],
            pltpu.SemaphoreType.DMA((_NBUF,)),
            pltpu.SemaphoreType.DMA,
        ],
        compiler_params=pltpu.CompilerParams(
            vmem_limit_bytes=64 * 1024 * 1024,
        ),
    )(x, adj, W, b2)
